# baseline (speedup 1.0000x reference)
import functools

import jax
import jax.numpy as jnp
from jax import lax
from jax.experimental import pallas as pl
from jax.experimental.pallas import tpu as pltpu
from jax.experimental.pallas import tpu_sc as plsc

N = 10000
E = 320000
D_IN = 128
H = 64
G = 64

NC = 2
NS = 16
NW = NC * NS

NP = 10240
RPT = NP // NS
CH = 125
EPT = E // NW
NCHUNK = EPT // CH
ER = E // CH
NB = 8
LAG = 2

_mesh = plsc.VectorSubcoreMesh(core_axis_name="c", subcore_axis_name="s")


@functools.partial(
    pl.kernel,
    out_type=jax.ShapeDtypeStruct((NC * NP,), jnp.float32),
    mesh=_mesh,
    scratch_types=[
        pltpu.VMEM((NCHUNK, CH), jnp.int32),
        pltpu.VMEM((RPT,), jnp.float32),
        pltpu.VMEM_SHARED((NP,), jnp.float32),
        pltpu.SemaphoreType.DMA((NB,)),
    ],
)
def _deg_kernel(dst_hbm, out_hbm, idx_v, ones_v, deg_sh, ssem):
    c = lax.axis_index("c")
    s = lax.axis_index("s")
    wid = c * NS + s

    one16 = jnp.full((16,), 1.0, jnp.float32)
    for i in range(RPT // 16):
        ones_v[pl.ds(i * 16, 16)] = one16

    pltpu.sync_copy(ones_v, deg_sh.at[pl.ds(s * RPT, RPT)])
    pltpu.sync_copy(dst_hbm.at[pl.ds(wid * NCHUNK, NCHUNK)], idx_v)
    plsc.subcore_barrier()

    def step(t, carry):
        for b in range(NB):
            j = t * NB + b

            @pl.when(t >= 1)
            def _():
                pltpu.make_async_copy(
                    ones_v.at[pl.ds(0, CH)], deg_sh.at[idx_v.at[0]],
                    ssem.at[b]).wait()

            pltpu.async_copy(ones_v.at[pl.ds(0, CH)], deg_sh.at[idx_v.at[j]],
                             ssem.at[b], add=True)
        return carry

    lax.fori_loop(0, NCHUNK // NB, step, 0)
    for b in range(NB):
        pltpu.make_async_copy(ones_v.at[pl.ds(0, CH)], deg_sh.at[idx_v.at[0]],
                              ssem.at[b]).wait()
    plsc.subcore_barrier()
    pltpu.sync_copy(deg_sh.at[pl.ds(s * RPT, RPT)],
                    out_hbm.at[pl.ds(c * NP + s * RPT, RPT)])


@functools.partial(
    pl.kernel,
    out_type=jax.ShapeDtypeStruct((NC, NP, H), jnp.bfloat16),
    mesh=_mesh,
    scratch_types=[
        pltpu.VMEM((NCHUNK, CH), jnp.int32),
        pltpu.VMEM((NCHUNK, CH), jnp.int32),
        pltpu.VMEM((NB, CH, H), jnp.bfloat16),
        pltpu.VMEM_SHARED((NP, H), jnp.bfloat16),
        pltpu.SemaphoreType.DMA((NB,)),
        pltpu.SemaphoreType.DMA((NB,)),
    ],
    compiler_params=pltpu.CompilerParams(use_tc_tiling_on_sc=False),
)
def _agg_kernel(ht_hbm, zeros_hbm, src_hbm, dst_hbm, out_hbm, srcv, dstv, rows,
                acc_sh, gsem, ssem):
    c = lax.axis_index("c")
    s = lax.axis_index("s")
    wid = c * NS + s

    @pl.when(c == 0)
    def _():
        pltpu.sync_copy(ht_hbm.at[pl.ds(s * RPT, RPT)],
                        acc_sh.at[pl.ds(s * RPT, RPT)])

    @pl.when(c == 1)
    def _():
        pltpu.sync_copy(zeros_hbm, acc_sh.at[pl.ds(s * RPT, RPT)])
    pltpu.sync_copy(src_hbm.at[pl.ds(wid * NCHUNK, NCHUNK)], srcv)
    pltpu.sync_copy(dst_hbm.at[pl.ds(wid * NCHUNK, NCHUNK)], dstv)
    plsc.subcore_barrier()

    PRE = NB - LAG

    def gather(j, b):
        pltpu.async_copy(ht_hbm.at[srcv.at[j]], rows.at[b], gsem.at[b])

    def wait_gather(b):
        pltpu.make_async_copy(ht_hbm.at[srcv.at[0]], rows.at[b],
                              gsem.at[b]).wait()

    def scatter(j, b):
        pltpu.async_copy(rows.at[b], acc_sh.at[dstv.at[j]], ssem.at[b],
                         add=True)

    def wait_scatter(b):
        pltpu.make_async_copy(rows.at[b], acc_sh.at[dstv.at[0]],
                              ssem.at[b]).wait()

    for b in range(PRE):
        gather(b, b)
    for b in range(NB):
        bg = (b - LAG) % NB
        if b >= LAG:
            wait_scatter(bg)
        gather(b + PRE, bg)
        wait_gather(b)
        scatter(b, b)

    def step(t, carry):
        for b in range(NB):
            j = t * NB + b
            bg = (b - LAG) % NB
            wait_scatter(bg)
            gather(j + PRE, bg)
            wait_gather(b)
            scatter(j, b)
        return carry

    lax.fori_loop(1, NCHUNK // NB - 1, step, 0)

    for b in range(NB):
        j = NCHUNK - NB + b
        bg = (b - LAG) % NB
        if b < LAG:
            wait_scatter(bg)
            gather(j + PRE, bg)
        wait_gather(b)
        scatter(j, b)
    for b in range(NB):
        wait_scatter(b)

    plsc.subcore_barrier()
    pltpu.sync_copy(acc_sh.at[pl.ds(s * RPT, RPT)], out_hbm.at[c, pl.ds(s * RPT, RPT)])


def _tc_a1_body(x_ref, w1_ref, h_ref):
    h = jnp.dot(x_ref[...], w1_ref[...], preferred_element_type=jnp.float32)
    h_ref[0:N, :] = h
    h_ref[N:NP, :] = jnp.zeros((NP - N, H), jnp.float32)


_tc_a1 = pl.pallas_call(
    _tc_a1_body,
    out_shape=jax.ShapeDtypeStruct((NP, H), jnp.float32),
)


def _tc_a2_body(h_ref, degp_ref, dis_ref, ht_ref):
    deg = degp_ref[:, 0:1] + degp_ref[:, 1:2] - 1.0
    dis = lax.rsqrt(deg)
    dis_ref[...] = dis
    ht_ref[...] = (h_ref[...] * dis).astype(jnp.bfloat16)


_tc_a2 = pl.pallas_call(
    _tc_a2_body,
    out_shape=(
        jax.ShapeDtypeStruct((NP, 1), jnp.float32),
        jax.ShapeDtypeStruct((NP, H), jnp.bfloat16),
    ),
)


def _tc_c_body(p_ref, dis_ref, b_ref, w_ref, out_ref):
    acc = (p_ref[0].astype(jnp.float32)
           + p_ref[1].astype(jnp.float32))
    dis = dis_ref[...]
    h = jnp.maximum(acc * dis + b_ref[...], 0.0)
    valid = jnp.where(
        lax.broadcasted_iota(jnp.int32, (NP, 1), 0) < N, 1.0, 0.0)
    h = h * valid
    out_ref[...] = (jnp.dot(h, w_ref[...], preferred_element_type=jnp.float32)
                    * dis * valid).astype(jnp.bfloat16)


_tc_c = pl.pallas_call(
    _tc_c_body,
    out_shape=jax.ShapeDtypeStruct((NP, H), jnp.bfloat16),
)


def _tc_d_body(p_ref, dis_ref, b_ref, batch_ref, f1w_ref, f1b_ref,
               f2w_ref, f2b_ref, out_ref):
    acc = (p_ref[0].astype(jnp.float32)
           + p_ref[1].astype(jnp.float32))
    h3 = jnp.maximum(acc * dis_ref[...] + b_ref[...], 0.0)
    hv = h3[0:N, :]
    bt = batch_ref[...]
    gid = lax.broadcasted_iota(jnp.int32, (G, N), 0)
    P = jnp.where(bt == gid, 1.0, 0.0)
    sums = jnp.dot(P, hv, preferred_element_type=jnp.float32)
    counts = jnp.sum(P, axis=1, keepdims=True)
    pooled = sums / jnp.maximum(counts, 1.0)
    z = jnp.maximum(
        jnp.dot(pooled, f1w_ref[...], preferred_element_type=jnp.float32)
        + f1b_ref[...], 0.0)
    z2 = jnp.dot(z, f2w_ref[...], preferred_element_type=jnp.float32) + f2b_ref[...]
    out_ref[...] = 1.0 / (1.0 + jnp.exp(-z2))


_tc_d = pl.pallas_call(
    _tc_d_body,
    out_shape=jax.ShapeDtypeStruct((G, 1), jnp.float32),
)


def kernel(x, edge_index, batch, W1, b1, W2, b2, W3, b3, fc1_W, fc1_b, fc2_W, fc2_b):
    src = edge_index[0].reshape(ER, CH)
    dst = edge_index[1].reshape(ER, CH)
    zrows = jnp.zeros((RPT, H), jnp.bfloat16)

    h1f = _tc_a1(x, W1)
    degp = _deg_kernel(dst).reshape(NC, NP)
    dis, ht1 = _tc_a2(h1f, degp.T)

    p1 = _agg_kernel(ht1, zrows, src, dst)
    ht2 = _tc_c(p1, dis, b1.reshape(1, H), W2)

    p2 = _agg_kernel(ht2, zrows, src, dst)
    ht3 = _tc_c(p2, dis, b2.reshape(1, H), W3)

    p3 = _agg_kernel(ht3, zrows, src, dst)
    return _tc_d(p3, dis, b3.reshape(1, H), batch.reshape(1, N),
                 fc1_W, fc1_b.reshape(1, H // 2), fc2_W, fc2_b.reshape(1, 1))


# speedup vs baseline: 1.0273x; 1.0273x over previous
import functools

import jax
import jax.numpy as jnp
from jax import lax
from jax.experimental import pallas as pl
from jax.experimental.pallas import tpu as pltpu
from jax.experimental.pallas import tpu_sc as plsc

N = 10000
E = 320000
D_IN = 128
H = 64
G = 64

NC = 2
NS = 16
NW = NC * NS

NP = 10240
RPT = NP // NS
CH = 125
EPT = E // NW
NCHUNK = EPT // CH
ER = E // CH
NB = 10
LAG = 2

_mesh = plsc.VectorSubcoreMesh(core_axis_name="c", subcore_axis_name="s")


@functools.partial(
    pl.kernel,
    out_type=jax.ShapeDtypeStruct((NC * NP,), jnp.float32),
    mesh=_mesh,
    scratch_types=[
        pltpu.VMEM((NCHUNK, CH), jnp.int32),
        pltpu.VMEM((RPT,), jnp.float32),
        pltpu.VMEM_SHARED((NP,), jnp.float32),
        pltpu.SemaphoreType.DMA((NB,)),
    ],
)
def _deg_kernel(dst_hbm, out_hbm, idx_v, ones_v, deg_sh, ssem):
    c = lax.axis_index("c")
    s = lax.axis_index("s")
    wid = c * NS + s

    one16 = jnp.full((16,), 1.0, jnp.float32)
    for i in range(RPT // 16):
        ones_v[pl.ds(i * 16, 16)] = one16

    pltpu.sync_copy(ones_v, deg_sh.at[pl.ds(s * RPT, RPT)])
    pltpu.sync_copy(dst_hbm.at[pl.ds(wid * NCHUNK, NCHUNK)], idx_v)
    plsc.subcore_barrier()

    def step(t, carry):
        for b in range(NB):
            j = t * NB + b

            @pl.when(t >= 1)
            def _():
                pltpu.make_async_copy(
                    ones_v.at[pl.ds(0, CH)], deg_sh.at[idx_v.at[0]],
                    ssem.at[b]).wait()

            pltpu.async_copy(ones_v.at[pl.ds(0, CH)], deg_sh.at[idx_v.at[j]],
                             ssem.at[b], add=True)
        return carry

    lax.fori_loop(0, NCHUNK // NB, step, 0)
    for b in range(NB):
        pltpu.make_async_copy(ones_v.at[pl.ds(0, CH)], deg_sh.at[idx_v.at[0]],
                              ssem.at[b]).wait()
    plsc.subcore_barrier()
    pltpu.sync_copy(deg_sh.at[pl.ds(s * RPT, RPT)],
                    out_hbm.at[pl.ds(c * NP + s * RPT, RPT)])


@functools.partial(
    pl.kernel,
    out_type=jax.ShapeDtypeStruct((NC, NP, H), jnp.bfloat16),
    mesh=_mesh,
    scratch_types=[
        pltpu.VMEM((NCHUNK, CH), jnp.int32),
        pltpu.VMEM((NCHUNK, CH), jnp.int32),
        pltpu.VMEM((NB, CH, H), jnp.bfloat16),
        pltpu.VMEM_SHARED((NP, H), jnp.bfloat16),
        pltpu.SemaphoreType.DMA((NB,)),
        pltpu.SemaphoreType.DMA((NB,)),
    ],
    compiler_params=pltpu.CompilerParams(use_tc_tiling_on_sc=False),
)
def _agg_kernel(ht_hbm, zeros_hbm, src_hbm, dst_hbm, out_hbm, srcv, dstv, rows,
                acc_sh, gsem, ssem):
    c = lax.axis_index("c")
    s = lax.axis_index("s")
    wid = c * NS + s

    @pl.when(c == 0)
    def _():
        pltpu.sync_copy(ht_hbm.at[pl.ds(s * RPT, RPT)],
                        acc_sh.at[pl.ds(s * RPT, RPT)])

    @pl.when(c == 1)
    def _():
        pltpu.sync_copy(zeros_hbm, acc_sh.at[pl.ds(s * RPT, RPT)])
    pltpu.sync_copy(src_hbm.at[pl.ds(wid * NCHUNK, NCHUNK)], srcv)
    pltpu.sync_copy(dst_hbm.at[pl.ds(wid * NCHUNK, NCHUNK)], dstv)
    plsc.subcore_barrier()

    PRE = NB - LAG

    def gather(j, b):
        pltpu.async_copy(ht_hbm.at[srcv.at[j]], rows.at[b], gsem.at[b])

    def wait_gather(b):
        pltpu.make_async_copy(ht_hbm.at[srcv.at[0]], rows.at[b],
                              gsem.at[b]).wait()

    def scatter(j, b):
        pltpu.async_copy(rows.at[b], acc_sh.at[dstv.at[j]], ssem.at[b],
                         add=True)

    def wait_scatter(b):
        pltpu.make_async_copy(rows.at[b], acc_sh.at[dstv.at[0]],
                              ssem.at[b]).wait()

    for b in range(PRE):
        gather(b, b)
    for b in range(NB):
        bg = (b - LAG) % NB
        if b >= LAG:
            wait_scatter(bg)
        gather(b + PRE, bg)
        wait_gather(b)
        scatter(b, b)

    def step(t, carry):
        for b in range(NB):
            j = t * NB + b
            bg = (b - LAG) % NB
            wait_scatter(bg)
            gather(j + PRE, bg)
            wait_gather(b)
            scatter(j, b)
        return carry

    lax.fori_loop(1, NCHUNK // NB - 1, step, 0)

    for b in range(NB):
        j = NCHUNK - NB + b
        bg = (b - LAG) % NB
        if b < LAG:
            wait_scatter(bg)
            gather(j + PRE, bg)
        wait_gather(b)
        scatter(j, b)
    for b in range(NB):
        wait_scatter(b)

    plsc.subcore_barrier()
    pltpu.sync_copy(acc_sh.at[pl.ds(s * RPT, RPT)], out_hbm.at[c, pl.ds(s * RPT, RPT)])


def _tc_a_body(x_ref, w1_ref, degp_ref, dis_ref, ht_ref):
    deg = degp_ref[:, 0:1] + degp_ref[:, 1:2] - 1.0
    dis = lax.rsqrt(deg)
    dis_ref[...] = dis
    h = jnp.dot(x_ref[...], w1_ref[...], preferred_element_type=jnp.float32)
    ht_ref[0:N, :] = (h * dis[0:N]).astype(jnp.bfloat16)
    ht_ref[N:NP, :] = jnp.zeros((NP - N, H), jnp.bfloat16)


_tc_a = pl.pallas_call(
    _tc_a_body,
    out_shape=(
        jax.ShapeDtypeStruct((NP, 1), jnp.float32),
        jax.ShapeDtypeStruct((NP, H), jnp.bfloat16),
    ),
)


def _tc_c_body(p_ref, dis_ref, b_ref, w_ref, out_ref):
    acc = (p_ref[0].astype(jnp.float32)
           + p_ref[1].astype(jnp.float32))
    dis = dis_ref[...]
    h = jnp.maximum(acc * dis + b_ref[...], 0.0)
    valid = jnp.where(
        lax.broadcasted_iota(jnp.int32, (NP, 1), 0) < N, 1.0, 0.0)
    h = h * valid
    out_ref[...] = (jnp.dot(h, w_ref[...], preferred_element_type=jnp.float32)
                    * dis * valid).astype(jnp.bfloat16)


_tc_c = pl.pallas_call(
    _tc_c_body,
    out_shape=jax.ShapeDtypeStruct((NP, H), jnp.bfloat16),
)


def _tc_d_body(p_ref, dis_ref, b_ref, batch_ref, f1w_ref, f1b_ref,
               f2w_ref, f2b_ref, out_ref):
    acc = (p_ref[0].astype(jnp.float32)
           + p_ref[1].astype(jnp.float32))
    h3 = jnp.maximum(acc * dis_ref[...] + b_ref[...], 0.0)
    hv = h3[0:N, :]
    bt = batch_ref[...]
    gid = lax.broadcasted_iota(jnp.int32, (G, N), 0)
    P = jnp.where(bt == gid, 1.0, 0.0)
    sums = jnp.dot(P, hv, preferred_element_type=jnp.float32)
    counts = jnp.sum(P, axis=1, keepdims=True)
    pooled = sums / jnp.maximum(counts, 1.0)
    z = jnp.maximum(
        jnp.dot(pooled, f1w_ref[...], preferred_element_type=jnp.float32)
        + f1b_ref[...], 0.0)
    z2 = jnp.dot(z, f2w_ref[...], preferred_element_type=jnp.float32) + f2b_ref[...]
    out_ref[...] = 1.0 / (1.0 + jnp.exp(-z2))


_tc_d = pl.pallas_call(
    _tc_d_body,
    out_shape=jax.ShapeDtypeStruct((G, 1), jnp.float32),
)


def kernel(x, edge_index, batch, W1, b1, W2, b2, W3, b3, fc1_W, fc1_b, fc2_W, fc2_b):
    src = edge_index[0].reshape(ER, CH)
    dst = edge_index[1].reshape(ER, CH)
    zrows = jnp.zeros((RPT, H), jnp.bfloat16)

    degp = _deg_kernel(dst).reshape(NC, NP)
    dis, ht1 = _tc_a(x, W1, degp.T)

    p1 = _agg_kernel(ht1, zrows, src, dst)
    ht2 = _tc_c(p1, dis, b1.reshape(1, H), W2)

    p2 = _agg_kernel(ht2, zrows, src, dst)
    ht3 = _tc_c(p2, dis, b2.reshape(1, H), W3)

    p3 = _agg_kernel(ht3, zrows, src, dst)
    return _tc_d(p3, dis, b3.reshape(1, H), batch.reshape(1, N),
                 fc1_W, fc1_b.reshape(1, H // 2), fc2_W, fc2_b.reshape(1, 1))
